# Initial kernel scaffold; baseline (speedup 1.0000x reference)
import functools

import jax
import jax.numpy as jnp
from jax import lax
from jax.experimental import pallas as pl
from jax.experimental.pallas import tpu as pltpu
from jax.experimental.pallas import tpu_sc as plsc

_LANES = 16


def _permute_rows(n, d, c, rows_per_w, ch):
    mesh = plsc.VectorSubcoreMesh(core_axis_name="c", subcore_axis_name="s")
    nchunks = rows_per_w // ch

    @functools.partial(
        pl.kernel,
        mesh=mesh,
        out_type=jax.ShapeDtypeStruct((n, d), jnp.float32),
        scratch_types=[
            pltpu.VMEM((rows_per_w,), jnp.int32),
            pltpu.VMEM((ch, d), jnp.float32),
            pltpu.VMEM((ch, d), jnp.float32),
            pltpu.SemaphoreType.DMA,
            pltpu.SemaphoreType.DMA,
        ],
    )
    def k(z_hbm, idx_hbm, out_hbm, idx_v, buf0, buf1, sem_g, sem_w):
        wid = lax.axis_index("s") * 2 + lax.axis_index("c")
        row_base = wid * rows_per_w
        batch = row_base // c
        c0 = row_base - batch * c

        pltpu.sync_copy(idx_hbm.at[pl.ds(c0, rows_per_w)], idx_v)
        for i in range(rows_per_w // _LANES):
            sl = pl.ds(i * _LANES, _LANES)
            idx_v[sl] = idx_v[sl] + batch * c

        bufs = (buf0, buf1)

        def gather(j, buf):
            return pltpu.async_copy(
                z_hbm.at[idx_v.at[pl.ds(j * ch, ch)]], buf, sem_g
            )

        def put(j, buf):
            return pltpu.async_copy(
                buf, out_hbm.at[pl.ds(row_base + j * ch, ch)], sem_w
            )

        gather(0, bufs[0]).wait()
        for j in range(nchunks):
            if j + 1 < nchunks:
                g = gather(j + 1, bufs[(j + 1) % 2])
            w = put(j, bufs[j % 2])
            if j + 1 < nchunks:
                g.wait()
            w.wait()

    return k


def kernel(z, indices):
    b, c, d = z.shape
    n = b * c
    info = plsc.get_sparse_core_info()
    nw = info.num_cores * info.num_subcores
    rows_per_w = n // nw
    zf = z.reshape(n, d)
    out = _permute_rows(n, d, c, rows_per_w, ch=8)(zf, indices)
    return out.reshape(b, c, d)


# speedup vs baseline: 2.0924x; 2.0924x over previous
import functools

import jax
import jax.numpy as jnp
from jax import lax
from jax.experimental import pallas as pl
from jax.experimental.pallas import tpu as pltpu
from jax.experimental.pallas import tpu_sc as plsc


def _permute_rows(n, d, nw, nchunks, ch):
    mesh = plsc.VectorSubcoreMesh(core_axis_name="c", subcore_axis_name="s")
    rows_per_w = nchunks * ch

    @functools.partial(
        pl.kernel,
        mesh=mesh,
        out_type=jax.ShapeDtypeStruct((n, d), jnp.float32),
        scratch_types=[
            pltpu.VMEM((nchunks, ch), jnp.int32),
            pltpu.VMEM((ch, d), jnp.float32),
            pltpu.VMEM((ch, d), jnp.float32),
            pltpu.SemaphoreType.DMA,
            pltpu.SemaphoreType.DMA,
        ],
    )
    def k(z_hbm, idx_hbm, out_hbm, idx_v, buf0, buf1, sem_g, sem_w):
        wid = lax.axis_index("s") * 2 + lax.axis_index("c")
        row_base = wid * rows_per_w

        pltpu.sync_copy(idx_hbm.at[wid], idx_v)
        bufs = (buf0, buf1)

        def gather(j, buf):
            return pltpu.async_copy(z_hbm.at[idx_v.at[j]], buf, sem_g)

        def put(j, buf):
            return pltpu.async_copy(
                buf, out_hbm.at[pl.ds(row_base + j * ch, ch)], sem_w
            )

        gather(0, bufs[0]).wait()
        for j in range(nchunks):
            if j + 1 < nchunks:
                g = gather(j + 1, bufs[(j + 1) % 2])
            w = put(j, bufs[j % 2])
            if j + 1 < nchunks:
                g.wait()
            w.wait()

    return k


def kernel(z, indices):
    b, c, d = z.shape
    n = b * c
    info = plsc.get_sparse_core_info()
    nw = info.num_cores * info.num_subcores
    ch = 4
    nchunks = n // (nw * ch)
    row_idx = (jnp.arange(b, dtype=jnp.int32) * c)[:, None] + indices[None, :]
    row_idx = row_idx.reshape(nw, nchunks, ch)
    zf = z.reshape(n, d)
    out = _permute_rows(n, d, nw, nchunks, ch)(zf, row_idx)
    return out.reshape(b, c, d)


# speedup vs baseline: 2.1728x; 1.0384x over previous
import functools

import jax
import jax.numpy as jnp
from jax import lax
from jax.experimental import pallas as pl
from jax.experimental.pallas import tpu as pltpu
from jax.experimental.pallas import tpu_sc as plsc


def _permute_rows(n, d, nw, nchunks, ch):
    mesh = plsc.VectorSubcoreMesh(core_axis_name="c", subcore_axis_name="s")
    rows_per_w = nchunks * ch

    @functools.partial(
        pl.kernel,
        mesh=mesh,
        out_type=jax.ShapeDtypeStruct((n, d), jnp.float32),
        scratch_types=[
            pltpu.VMEM((nchunks, ch), jnp.int32),
            pltpu.VMEM((ch, d), jnp.float32),
            pltpu.VMEM((ch, d), jnp.float32),
            pltpu.VMEM((ch, d), jnp.float32),
            pltpu.SemaphoreType.DMA,
            pltpu.SemaphoreType.DMA,
        ],
    )
    def k(z_hbm, idx_hbm, out_hbm, idx_v, buf0, buf1, buf2, sem_g, sem_w):
        wid = lax.axis_index("s") * 2 + lax.axis_index("c")
        row_base = wid * rows_per_w

        pltpu.sync_copy(idx_hbm.at[wid], idx_v)
        bufs = (buf0, buf1, buf2)
        nb = len(bufs)

        def gather(j):
            return pltpu.async_copy(z_hbm.at[idx_v.at[j]], bufs[j % nb], sem_g)

        def put(j):
            return pltpu.async_copy(
                bufs[j % nb], out_hbm.at[pl.ds(row_base + j * ch, ch)], sem_w
            )

        gathers = [gather(j) for j in range(min(nb - 1, nchunks))]
        writes = [None] * nchunks
        for j in range(nchunks):
            if j + nb - 1 < nchunks:
                if j >= 1:
                    writes[j - 1].wait()
                gathers.append(gather(j + nb - 1))
            gathers[j].wait()
            writes[j] = put(j)
        for j in range(max(0, nchunks - nb), nchunks):
            writes[j].wait()

    return k


def kernel(z, indices):
    b, c, d = z.shape
    n = b * c
    info = plsc.get_sparse_core_info()
    nw = info.num_cores * info.num_subcores
    ch = 4
    nchunks = n // (nw * ch)
    row_idx = (jnp.arange(b, dtype=jnp.int32) * c)[:, None] + indices[None, :]
    row_idx = row_idx.reshape(nw, nchunks, ch)
    zf = z.reshape(n, d)
    out = _permute_rows(n, d, nw, nchunks, ch)(zf, row_idx)
    return out.reshape(b, c, d)
